# baseline (speedup 1.0000x reference)
import functools

import jax
import jax.numpy as jnp
from jax import lax
from jax.experimental import pallas as pl
from jax.experimental.pallas import tpu as pltpu
from jax.experimental.pallas import tpu_sc as plsc

VOCAB = 1000000
HIDDEN = 64
MAX_POS = 200
TYPE_VOCAB = 2
BATCH = 4096
SEQ = 200
EPS = 1e-12

L = 16
NC = 2
NS = 16
NW = NC * NS
TOK = BATCH * SEQ
TPW = TOK // NW
C = 512
NCH = TPW // C
JROWS = C // 128

_GDN = lax.GatherDimensionNumbers(
    offset_dims=(), collapsed_slice_dims=(0,), start_index_map=(0,))


def _splat_lane(v, lane):
    idx = jnp.full((L,), lane, jnp.int32)
    return lax.gather(v, idx[:, None], _GDN, (1,),
                      unique_indices=False, indices_are_sorted=False,
                      mode=lax.GatherScatterMode.PROMISE_IN_BOUNDS)


def _rsqrt_f32(x):
    i = lax.bitcast_convert_type(x, jnp.int32)
    i = jnp.int32(0x5F3759DF) - lax.shift_right_logical(i, 1)
    y = lax.bitcast_convert_type(i, jnp.float32)
    for _ in range(4):
        y = y * (1.5 - 0.5 * x * y * y)
    return y


def _body(ids_hbm, pos_hbm, typ_hbm, word_hbm, combo_hbm, gb_hbm, out_hbm,
          idx_v, pids_v, tids_v, rows_v, ebuf_v, combo_v, gb_v, sem):
    cid = lax.axis_index("c")
    sid = lax.axis_index("s")
    wid = sid * NC + cid
    base = wid * TPW
    row_base = wid * (TPW // 128)

    pltpu.sync_copy(combo_hbm, combo_v)
    pltpu.sync_copy(gb_hbm, gb_v)
    g = [gb_v[0, pl.ds(k * L, L)] for k in range(4)]
    b = [gb_v[1, pl.ds(k * L, L)] for k in range(4)]
    gs = [_splat_lane(g[h // L], h % L) for h in range(HIDDEN)]
    bs = [_splat_lane(b[h // L], h % L) for h in range(HIDDEN)]
    lane_iota = lax.iota(jnp.int32, L)

    def chunk(ci, carry):
        tb = base + ci * C
        pltpu.sync_copy(ids_hbm.at[pl.ds(row_base + ci * JROWS, JROWS)],
                        idx_v)
        pltpu.sync_copy(pos_hbm.at[pl.ds(tb, C)], pids_v)
        pltpu.sync_copy(typ_hbm.at[pl.ds(tb, C)], tids_v)
        cps = [pltpu.async_copy(word_hbm.at[idx_v.at[j]],
                                rows_v.at[pl.ds(j * 128, 128)], sem)
               for j in range(JROWS)]
        for cp in cps:
            cp.wait()

        def group(gi, acc):
            t0 = gi * L
            tvec = t0 + lane_iota
            pvec = pids_v[pl.ds(t0, L)]
            tyv = tids_v[pl.ds(t0, L)]
            cvec = pvec * 2 + tyv
            s = jnp.zeros((L,), jnp.float32)
            q = jnp.zeros((L,), jnp.float32)
            for h in range(HIDDEN):
                hv = jnp.full((L,), h, jnp.int32)
                w = plsc.load_gather(rows_v, [tvec, hv])
                c = plsc.load_gather(combo_v, [cvec, hv])
                e = w + c
                s = s + e
                q = q + e * e
                ebuf_v[h] = e
            mean = s * (1.0 / HIDDEN)
            var = q * (1.0 / HIDDEN) - mean * mean
            rs = _rsqrt_f32(var + EPS)
            c0 = -mean * rs
            for h in range(HIDDEN):
                hv = jnp.full((L,), h, jnp.int32)
                o = (ebuf_v[h] * rs + c0) * gs[h] + bs[h]
                plsc.store_scatter(rows_v, [tvec, hv], o)
            return acc

        lax.fori_loop(0, C // L, group, 0)
        pltpu.sync_copy(rows_v, out_hbm.at[pl.ds(tb, C)])
        return carry

    lax.fori_loop(0, NCH, chunk, 0)


@jax.jit
def _run(ids2d, pos_flat, typ_flat, word_table, combo, gb):
    mesh = plsc.VectorSubcoreMesh(core_axis_name="c", subcore_axis_name="s")
    f = pl.kernel(
        _body,
        out_type=jax.ShapeDtypeStruct((TOK, HIDDEN), jnp.float32),
        mesh=mesh,
        scratch_types=[
            pltpu.VMEM((JROWS, 128), jnp.int32),
            pltpu.VMEM((C,), jnp.int32),
            pltpu.VMEM((C,), jnp.int32),
            pltpu.VMEM((C, HIDDEN), jnp.float32),
            pltpu.VMEM((HIDDEN, L), jnp.float32),
            pltpu.VMEM((MAX_POS * TYPE_VOCAB, HIDDEN), jnp.float32),
            pltpu.VMEM((2, HIDDEN), jnp.float32),
            pltpu.SemaphoreType.DMA,
        ],
        compiler_params=pltpu.CompilerParams(use_tc_tiling_on_sc=False,
                                             needs_layout_passes=False),
    )
    return f(ids2d, pos_flat, typ_flat, word_table, combo, gb)


def kernel(input_ids, position_ids, token_type_ids, word_table, pos_table,
           type_table, ln_gamma, ln_beta):
    ids2d = input_ids.reshape(TOK // 128, 128).astype(jnp.int32)
    pos_flat = position_ids.reshape(TOK).astype(jnp.int32)
    typ_flat = token_type_ids.reshape(TOK).astype(jnp.int32)
    combo = (pos_table[:, None, :] + type_table[None, :, :]).reshape(
        MAX_POS * TYPE_VOCAB, HIDDEN)
    gb = jnp.stack([ln_gamma, ln_beta]).astype(jnp.float32)
    out = _run(ids2d, pos_flat, typ_flat, word_table, combo, gb)
    return out.reshape(BATCH, SEQ, HIDDEN)


# speedup vs baseline: 2.4736x; 2.4736x over previous
import functools

import jax
import jax.numpy as jnp
from jax import lax
from jax.experimental import pallas as pl
from jax.experimental.pallas import tpu as pltpu
from jax.experimental.pallas import tpu_sc as plsc

VOCAB = 1000000
HIDDEN = 64
MAX_POS = 200
TYPE_VOCAB = 2
BATCH = 4096
SEQ = 200
EPS = 1e-12

L = 16
NC = 2
NS = 16
NW = NC * NS
TOK = BATCH * SEQ
TPW = TOK // NW
C = 256
NCH = TPW // C
JROWS = C // 128
NBUF = 3

_GDN = lax.GatherDimensionNumbers(
    offset_dims=(), collapsed_slice_dims=(0,), start_index_map=(0,))


def _lane_perms():
    iota = lax.iota(jnp.int32, L)
    return [jnp.bitwise_xor(iota, jnp.int32(d)) for d in (1, 2, 4, 8)]


def _lanesum(v, perms):
    for p in perms:
        v = v + lax.gather(v, p[:, None], _GDN, (1,),
                           unique_indices=True, indices_are_sorted=False,
                           mode=lax.GatherScatterMode.PROMISE_IN_BOUNDS)
    return v


def _rsqrt_f32(x):
    i = lax.bitcast_convert_type(x, jnp.int32)
    i = jnp.int32(0x5F3759DF) - lax.shift_right_logical(i, 1)
    y = lax.bitcast_convert_type(i, jnp.float32)
    for _ in range(2):
        y = y * (1.5 - 0.5 * x * y * y)
    return y


def _body(ids_hbm, cids_hbm, word_hbm, combo_hbm, gb_hbm, out_hbm,
          idx_v, cids_v, rows_v, combo_v, gb_v,
          gsem0, gsem1, gsem2, wsem0, wsem1, wsem2):
    gsems = [gsem0, gsem1, gsem2]
    wsems = [wsem0, wsem1, wsem2]
    cid_ = lax.axis_index("c")
    sid = lax.axis_index("s")
    wid = sid * NC + cid_
    base = wid * TPW
    row_base = wid * (TPW // 128)

    pltpu.sync_copy(combo_hbm, combo_v)
    pltpu.sync_copy(gb_hbm, gb_v)
    g = [gb_v[0, pl.ds(k * L, L)] for k in range(4)]
    b = [gb_v[1, pl.ds(k * L, L)] for k in range(4)]
    perms = _lane_perms()

    def fire(X, bf):
        tb = base + X * C
        pltpu.sync_copy(ids_hbm.at[pl.ds(row_base + X * JROWS, JROWS)],
                        idx_v.at[pl.ds(bf * JROWS, JROWS)])
        pltpu.async_copy(cids_hbm.at[pl.ds(tb, C)],
                         cids_v.at[pl.ds(bf * C, C)], gsems[bf])
        for j in range(JROWS):
            pltpu.async_copy(word_hbm.at[idx_v.at[bf * JROWS + j]],
                             rows_v.at[pl.ds(bf * C + j * 128, 128)],
                             gsems[bf])

    def drain(bf):
        pltpu.make_async_copy(cids_hbm.at[pl.ds(0, C)],
                              cids_v.at[pl.ds(bf * C, C)],
                              gsems[bf]).wait()
        for j in range(JROWS):
            pltpu.make_async_copy(word_hbm.at[idx_v.at[bf * JROWS + j]],
                                  rows_v.at[pl.ds(bf * C + j * 128, 128)],
                                  gsems[bf]).wait()

    def fire_wb(X, bf):
        tb = base + X * C
        pltpu.async_copy(rows_v.at[pl.ds(bf * C, C)],
                         out_hbm.at[pl.ds(tb, C)], wsems[bf])

    def wait_wb(bf):
        pltpu.make_async_copy(rows_v.at[pl.ds(bf * C, C)],
                              out_hbm.at[pl.ds(0, C)], wsems[bf]).wait()

    def compute(bf):
        rofs = bf * C

        def group(gi, acc):
            t0 = rofs + gi * L
            cv = cids_v[pl.ds(t0, L)]
            for u in range(L):
                t = t0 + u
                ci = cv[u]
                e = [rows_v[t, pl.ds(k * L, L)]
                     + combo_v[ci, pl.ds(k * L, L)]
                     for k in range(4)]
                s = _lanesum(e[0] + e[1] + e[2] + e[3], perms)
                q = _lanesum(e[0] * e[0] + e[1] * e[1]
                             + e[2] * e[2] + e[3] * e[3], perms)
                mean = s * (1.0 / HIDDEN)
                var = q * (1.0 / HIDDEN) - mean * mean
                rs = _rsqrt_f32(var + EPS)
                c0 = -mean * rs
                for k in range(4):
                    rows_v[t, pl.ds(k * L, L)] = (e[k] * rs + c0) * g[k] + b[k]
            return acc

        lax.fori_loop(0, C // L, group, 0)

    def step(X, bf, wait_prev=True, ahead=True):
        drain(bf)
        compute(bf)
        fire_wb(X, bf)
        nb = (bf + 2) % NBUF
        if wait_prev:
            wait_wb(nb)
        if ahead:
            fire(X + 2, nb)

    fire(0, 0)
    fire(1, 1)
    step(0, 0, wait_prev=False)
    step(1, 1)

    def iter3(i, carry):
        X = 2 + 3 * i
        step(X, 2)
        step(X + 1, 0)
        step(X + 2, 1)
        return carry

    lax.fori_loop(0, (NCH - 4) // 3, iter3, 0)
    step(NCH - 2, (NCH - 2) % NBUF, ahead=False)
    step(NCH - 1, (NCH - 1) % NBUF, ahead=False)
    wait_wb((NCH - 1) % NBUF)


@jax.jit
def _run(ids2d, cids_flat, word_table, combo, gb):
    mesh = plsc.VectorSubcoreMesh(core_axis_name="c", subcore_axis_name="s")
    f = pl.kernel(
        _body,
        out_type=jax.ShapeDtypeStruct((TOK, HIDDEN), jnp.float32),
        mesh=mesh,
        scratch_types=[
            pltpu.VMEM((NBUF * JROWS, 128), jnp.int32),
            pltpu.VMEM((NBUF * C,), jnp.int32),
            pltpu.VMEM((NBUF * C, HIDDEN), jnp.float32),
            pltpu.VMEM((MAX_POS * TYPE_VOCAB, HIDDEN), jnp.float32),
            pltpu.VMEM((2, HIDDEN), jnp.float32),
            pltpu.SemaphoreType.DMA,
            pltpu.SemaphoreType.DMA,
            pltpu.SemaphoreType.DMA,
            pltpu.SemaphoreType.DMA,
            pltpu.SemaphoreType.DMA,
            pltpu.SemaphoreType.DMA,
        ],
        compiler_params=pltpu.CompilerParams(use_tc_tiling_on_sc=False,
                                             needs_layout_passes=False),
    )
    return f(ids2d, cids_flat, word_table, combo, gb)


def kernel(input_ids, position_ids, token_type_ids, word_table, pos_table,
           type_table, ln_gamma, ln_beta):
    ids2d = input_ids.reshape(TOK // 128, 128).astype(jnp.int32)
    cids = (position_ids.astype(jnp.int32) * TYPE_VOCAB
            + token_type_ids.astype(jnp.int32)).reshape(TOK)
    combo = (pos_table[:, None, :] + type_table[None, :, :]).reshape(
        MAX_POS * TYPE_VOCAB, HIDDEN)
    gb = jnp.stack([ln_gamma, ln_beta]).astype(jnp.float32)
    out = _run(ids2d, cids, word_table, combo, gb)
    return out.reshape(BATCH, SEQ, HIDDEN)


# speedup vs baseline: 4.3283x; 1.7498x over previous
import functools

import jax
import jax.numpy as jnp
from jax import lax
from jax.experimental import pallas as pl
from jax.experimental.pallas import tpu as pltpu
from jax.experimental.pallas import tpu_sc as plsc

VOCAB = 1000000
HIDDEN = 64
MAX_POS = 200
TYPE_VOCAB = 2
BATCH = 4096
SEQ = 200
EPS = 1e-12

L = 16
NC = 2
NS = 16
NW = NC * NS
TOK = BATCH * SEQ
TPW = TOK // NW
C = 256
NCH = TPW // C
JROWS = C // 128
NBUF = 3

_GDN = lax.GatherDimensionNumbers(
    offset_dims=(), collapsed_slice_dims=(0,), start_index_map=(0,))


def _lane_perms():
    iota = lax.iota(jnp.int32, L)
    return [jnp.bitwise_xor(iota, jnp.int32(d)) for d in (1, 2, 4, 8)]


def _lanesum(v, perms):
    for p in perms:
        v = v + lax.gather(v, p[:, None], _GDN, (1,),
                           unique_indices=True, indices_are_sorted=False,
                           mode=lax.GatherScatterMode.PROMISE_IN_BOUNDS)
    return v


def _rsqrt_f32(x):
    i = lax.bitcast_convert_type(x, jnp.int32)
    i = jnp.int32(0x5F3759DF) - lax.shift_right_logical(i, 1)
    y = lax.bitcast_convert_type(i, jnp.float32)
    for _ in range(2):
        y = y * (1.5 - 0.5 * x * y * y)
    return y


def _body(ids_hbm, cids_hbm, word_hbm, combo_hbm, gb_hbm, out_hbm,
          idx_v, cids_v, rows_v, obuf_v, combo_v, gb_v,
          gsem0, gsem1, gsem2, wsem0, wsem1, wsem2):
    gsems = [gsem0, gsem1, gsem2]
    wsems = [wsem0, wsem1, wsem2]
    cid_ = lax.axis_index("c")
    sid = lax.axis_index("s")
    wid = sid * NC + cid_
    base = wid * TPW
    row_base = wid * (TPW // 128)

    pltpu.sync_copy(combo_hbm, combo_v)
    pltpu.sync_copy(gb_hbm, gb_v)
    g = [gb_v[0, pl.ds(k * L, L)] for k in range(4)]
    b = [gb_v[1, pl.ds(k * L, L)] for k in range(4)]
    perms = _lane_perms()

    def fire(X, bf):
        tb = base + X * C
        pltpu.sync_copy(ids_hbm.at[pl.ds(row_base + X * JROWS, JROWS)],
                        idx_v.at[pl.ds(bf * JROWS, JROWS)])
        pltpu.async_copy(cids_hbm.at[pl.ds(tb, C)],
                         cids_v.at[pl.ds(bf * C, C)], gsems[bf])
        for j in range(JROWS):
            pltpu.async_copy(word_hbm.at[idx_v.at[bf * JROWS + j]],
                             rows_v.at[pl.ds(bf * C + j * 128, 128)],
                             gsems[bf])

    def drain(bf):
        pltpu.make_async_copy(cids_hbm.at[pl.ds(0, C)],
                              cids_v.at[pl.ds(bf * C, C)],
                              gsems[bf]).wait()
        for j in range(JROWS):
            pltpu.make_async_copy(word_hbm.at[idx_v.at[bf * JROWS + j]],
                                  rows_v.at[pl.ds(bf * C + j * 128, 128)],
                                  gsems[bf]).wait()

    def fire_wb(X, bf):
        tb = base + X * C
        pltpu.async_copy(obuf_v.at[pl.ds(bf * C, C)],
                         out_hbm.at[pl.ds(tb, C)], wsems[bf])

    def wait_wb(bf):
        pltpu.make_async_copy(obuf_v.at[pl.ds(bf * C, C)],
                              out_hbm.at[pl.ds(0, C)], wsems[bf]).wait()

    def compute(bf):
        rofs = bf * C

        @functools.partial(plsc.parallel_loop, 0, C // L, unroll=2)
        def group(gi):
            t0 = rofs + gi * L
            cv = cids_v[pl.ds(t0, L)]
            for u in range(L):
                t = t0 + u
                ci = cv[u]
                e = [rows_v[t, pl.ds(k * L, L)]
                     + combo_v[ci, pl.ds(k * L, L)]
                     for k in range(4)]
                s = _lanesum(e[0] + e[1] + e[2] + e[3], perms)
                q = _lanesum(e[0] * e[0] + e[1] * e[1]
                             + e[2] * e[2] + e[3] * e[3], perms)
                mean = s * (1.0 / HIDDEN)
                var = q * (1.0 / HIDDEN) - mean * mean
                rs = _rsqrt_f32(var + EPS)
                c0 = -mean * rs
                for k in range(4):
                    obuf_v[t, pl.ds(k * L, L)] = (e[k] * rs + c0) * g[k] + b[k]

    def step(X, bf, wait_prev=True, ahead=True):
        drain(bf)
        compute(bf)
        fire_wb(X, bf)
        nb = (bf + 2) % NBUF
        if wait_prev:
            wait_wb(nb)
        if ahead:
            fire(X + 2, nb)

    fire(0, 0)
    fire(1, 1)
    step(0, 0, wait_prev=False)
    step(1, 1)

    def iter3(i, carry):
        X = 2 + 3 * i
        step(X, 2)
        step(X + 1, 0)
        step(X + 2, 1)
        return carry

    lax.fori_loop(0, (NCH - 4) // 3, iter3, 0)
    step(NCH - 2, (NCH - 2) % NBUF, ahead=False)
    step(NCH - 1, (NCH - 1) % NBUF, ahead=False)
    wait_wb((NCH - 1) % NBUF)


@jax.jit
def _run(ids2d, cids_flat, word_table, combo, gb):
    mesh = plsc.VectorSubcoreMesh(core_axis_name="c", subcore_axis_name="s")
    f = pl.kernel(
        _body,
        out_type=jax.ShapeDtypeStruct((TOK, HIDDEN), jnp.float32),
        mesh=mesh,
        scratch_types=[
            pltpu.VMEM((NBUF * JROWS, 128), jnp.int32),
            pltpu.VMEM((NBUF * C,), jnp.int32),
            pltpu.VMEM((NBUF * C, HIDDEN), jnp.float32),
            pltpu.VMEM((NBUF * C, HIDDEN), jnp.float32),
            pltpu.VMEM((MAX_POS * TYPE_VOCAB, HIDDEN), jnp.float32),
            pltpu.VMEM((2, HIDDEN), jnp.float32),
            pltpu.SemaphoreType.DMA,
            pltpu.SemaphoreType.DMA,
            pltpu.SemaphoreType.DMA,
            pltpu.SemaphoreType.DMA,
            pltpu.SemaphoreType.DMA,
            pltpu.SemaphoreType.DMA,
        ],
        compiler_params=pltpu.CompilerParams(use_tc_tiling_on_sc=False,
                                             needs_layout_passes=False),
    )
    return f(ids2d, cids_flat, word_table, combo, gb)


def kernel(input_ids, position_ids, token_type_ids, word_table, pos_table,
           type_table, ln_gamma, ln_beta):
    ids2d = input_ids.reshape(TOK // 128, 128).astype(jnp.int32)
    cids = (position_ids.astype(jnp.int32) * TYPE_VOCAB
            + token_type_ids.astype(jnp.int32)).reshape(TOK)
    combo = (pos_table[:, None, :] + type_table[None, :, :]).reshape(
        MAX_POS * TYPE_VOCAB, HIDDEN)
    gb = jnp.stack([ln_gamma, ln_beta]).astype(jnp.float32)
    out = _run(ids2d, cids, word_table, combo, gb)
    return out.reshape(BATCH, SEQ, HIDDEN)
